# Initial kernel scaffold; baseline (speedup 1.0000x reference)
import jax
import jax.numpy as jnp
from jax.experimental import pallas as pl


def kernel(pred_points, points, masks):
    raise NotImplementedError("write your pallas kernel here")


# speedup vs baseline: 7.3809x; 7.3809x over previous
import functools

import jax
import jax.numpy as jnp
from jax import lax
from jax.experimental import pallas as pl
from jax.experimental.pallas import tpu as pltpu
from jax.experimental.pallas import tpu_sc as plsc

W_DIRECT = 1.0
W_MULTIPLE = 1.0
W_BACKGROUND = 1.0

NC = 2
NS = 16
L = 16
NW = NC * NS

BIG = 1e30
BIGH = 1e29
IBIG = 2**30


def _make_sc_kernel(N, NPAD, M, MPAD, H, W, KROWS):
    HW = H * W
    CHUNKS = NPAD // L
    REAL_CHUNKS = N // L
    KMAX = (M + NW - 1) // NW
    mesh = plsc.VectorSubcoreMesh(core_axis_name="c", subcore_axis_name="s")

    @functools.partial(
        pl.kernel,
        mesh=mesh,
        compiler_params=pltpu.CompilerParams(needs_layout_passes=False),
        out_type=[
            jax.ShapeDtypeStruct((NW, KROWS, L), jnp.float32),
            jax.ShapeDtypeStruct((NW, KROWS, L), jnp.float32),
            jax.ShapeDtypeStruct((NW, KROWS, L), jnp.int32),
            jax.ShapeDtypeStruct((NW, NPAD), jnp.float32),
        ],
        scratch_types=[
            pltpu.VMEM((NPAD,), jnp.float32),
            pltpu.VMEM((NPAD,), jnp.float32),
            pltpu.VMEM((NPAD,), jnp.int32),
            pltpu.VMEM((NPAD,), jnp.int32),
            pltpu.VMEM((NPAD,), jnp.float32),
            pltpu.VMEM((NPAD,), jnp.float32),
            pltpu.VMEM((NW, L), jnp.float32),
            pltpu.VMEM((KROWS, L), jnp.float32),
            pltpu.VMEM((KROWS, L), jnp.float32),
            pltpu.VMEM((KROWS, L), jnp.int32),
            pltpu.VMEM((L,), jnp.float32),
            pltpu.VMEM((L,), jnp.int32),
            pltpu.VMEM((L,), jnp.float32),
            pltpu.SemaphoreType.DMA,
        ],
    )
    def sc_kernel(masks_hbm, ux_hbm, uy_hbm, v_hbm,
                  cnt_o, d2_o, idx_o, any_o,
                  ux_v, uy_v, pbase_v, offs_v, vals_v, any_v, v_v,
                  cntb, d2b, idxb, tmpd, tmpi, tmpc, sem):
        wid = lax.axis_index("s") * NC + lax.axis_index("c")
        pltpu.sync_copy(ux_hbm, ux_v)
        pltpu.sync_copy(uy_hbm, uy_v)
        pltpu.sync_copy(v_hbm, v_v)

        zero16f = jnp.zeros((L,), jnp.float32)
        zero16i = jnp.zeros((L,), jnp.int32)

        def init_body(i, c):
            sl = pl.ds(i * L, L)
            any_v[sl] = zero16f
            xi = jnp.clip(ux_v[sl].astype(jnp.int32), 0, W - 1)
            yi = jnp.clip(uy_v[sl].astype(jnp.int32), 0, H - 1)
            pbase_v[sl] = yi * W + xi
            return c
        lax.fori_loop(0, CHUNKS, init_body, 0)

        for k in range(KROWS):
            cntb[k, :] = zero16f
            d2b[k, :] = zero16f
            idxb[k, :] = zero16i

        base_iota = lax.iota(jnp.int32, L)
        vvec = v_v[wid, :]

        for k in range(KMAX):
            m = k * NW + wid

            @pl.when(m < M)
            def _():
                mhw = m * HW

                def offs_body(i, c):
                    sl = pl.ds(i * L, L)
                    offs_v[sl] = pbase_v[sl] + mhw
                    return c
                lax.fori_loop(0, CHUNKS, offs_body, 0)

                pltpu.async_copy(masks_hbm.at[offs_v], vals_v, sem).wait()

                vx = vvec[k]
                vy = vvec[KROWS + k]

                def chunk_body(i, carry):
                    bd2, bidx, cnt = carry
                    sl = pl.ds(i * L, L)
                    val = vals_v[sl]
                    member = val > 0.0
                    dx = ux_v[sl] - vx
                    dy = uy_v[sl] - vy
                    d2 = dx * dx + dy * dy
                    cand = jnp.where(member, d2, BIG)
                    lt = cand < bd2
                    idxv = base_iota + i * L
                    bd2 = jnp.where(lt, cand, bd2)
                    bidx = jnp.where(lt, idxv, bidx)
                    cnt = cnt + jnp.where(member, 1.0, 0.0)
                    any_v[sl] = jnp.maximum(
                        any_v[sl], jnp.where(member, 1.0, 0.0))
                    return (bd2, bidx, cnt)

                bd2, bidx, cnt = lax.fori_loop(
                    0, REAL_CHUNKS, chunk_body,
                    (jnp.full((L,), BIG, jnp.float32), zero16i, zero16f))

                tmpd[:] = bd2
                tmpi[:] = bidx
                tmpc[:] = cnt
                for sh in (8, 4, 2, 1):
                    perm = jnp.bitwise_xor(base_iota, sh)
                    od2 = plsc.load_gather(tmpd, [perm])
                    oidx = plsc.load_gather(tmpi, [perm])
                    oc = plsc.load_gather(tmpc, [perm])
                    d2v = tmpd[:]
                    idxv = tmpi[:]
                    better = (od2 < d2v) | ((od2 == d2v) & (oidx < idxv))
                    tmpd[:] = jnp.where(better, od2, d2v)
                    tmpi[:] = jnp.where(better, oidx, idxv)
                    tmpc[:] = tmpc[:] + oc
                cntb[k, :] = tmpc[:]
                d2b[k, :] = tmpd[:]
                idxb[k, :] = tmpi[:]

        pltpu.sync_copy(cntb, cnt_o.at[wid])
        pltpu.sync_copy(d2b, d2_o.at[wid])
        pltpu.sync_copy(idxb, idx_o.at[wid])
        pltpu.sync_copy(any_v, any_o.at[wid])

    return sc_kernel


def _make_tc_kernel(N, NPAD, M, MPAD, NROW):
    NCOL = NPAD // NROW

    def tc_body(cnt_s, vx_s, vy_s, cnt_v, d2_v, idx_v, any_r, ux_r, uy_r,
                pairs_o, cost_o):
        ux = ux_r[...]
        uy = uy_r[...]
        rr = lax.broadcasted_iota(jnp.int32, (NROW, NCOL), 0)
        cc = lax.broadcasted_iota(jnp.int32, (NROW, NCOL), 1)
        nidx = rr * NCOL + cc

        anyacc = any_r[0]
        for i in range(1, NW):
            anyacc = anyacc + any_r[i]
        avail = jnp.where((anyacc == 0.0) & (nidx < N), 1.0, 0.0)

        miota = lax.broadcasted_iota(jnp.int32, (1, MPAD), 1)
        cntv = cnt_v[...]
        validm = (cntv > 0.0) & (miota < M)
        wgt = jnp.where(cntv == 1.0, W_DIRECT, W_MULTIPLE).astype(jnp.float32)
        cost0 = jnp.sum(jnp.where(validm, jnp.sqrt(d2_v[...]) * wgt, 0.0))
        pairs = jnp.where(validm, idx_v[...], -1)
        nempty = jnp.sum(jnp.where((cntv <= 0.0) & (miota < M), 1, 0))

        def bg_body(j, st):
            availf, prs, cst = st
            cj = cnt_s[j]
            vxj = vx_s[j]
            vyj = vy_s[j]
            dx = ux - vxj
            dy = uy - vyj
            ds2 = dx * dx + dy * dy
            cand = jnp.where(availf > 0.0, ds2, BIG)
            mv = jnp.min(cand)
            do = (cj == 0.0) & (mv < BIGH)
            mi = jnp.min(jnp.where(cand == mv, nidx, IBIG))
            prs = jnp.where((miota == j) & do, mi, prs)
            cst = cst + jnp.where(do, jnp.sqrt(mv) * W_BACKGROUND, 0.0)
            availf = jnp.where((nidx == mi) & do, 0.0, availf)
            return (availf, prs, cst)

        st = (avail, pairs, jnp.float32(0.0))
        _, pairs, bgcost = lax.cond(
            nempty > 0, lambda s: lax.fori_loop(0, M, bg_body, s),
            lambda s: s, st)

        pairs_o[...] = pairs
        cost_o[...] = (cost0 + bgcost).reshape(1, 1)

    smem = pl.BlockSpec(memory_space=pltpu.SMEM)
    vmem = pl.BlockSpec(memory_space=pltpu.VMEM)
    return pl.pallas_call(
        tc_body,
        in_specs=[smem, smem, smem, vmem, vmem, vmem, vmem, vmem, vmem],
        out_specs=[vmem, vmem],
        out_shape=[
            jax.ShapeDtypeStruct((1, MPAD), jnp.int32),
            jax.ShapeDtypeStruct((1, 1), jnp.float32),
        ],
    )


def kernel(pred_points, points, masks):
    N = pred_points.shape[0]
    M, H, W = masks.shape

    NPAD = ((N + NW * L - 1) // (NW * L)) * (NW * L)
    MPAD = 256
    KROWS = 8
    NROW = 8

    ux = jnp.pad(pred_points[:, 0], (0, NPAD - N))
    uy = jnp.pad(pred_points[:, 1], (0, NPAD - N))
    v2 = jnp.pad(points, ((0, MPAD - M), (0, 0)))
    vw = jnp.concatenate(
        [v2[:, 0].reshape(KROWS, NW).T, v2[:, 1].reshape(KROWS, NW).T],
        axis=1)
    masks_flat = masks.reshape(-1)

    sc = _make_sc_kernel(N, NPAD, M, MPAD, H, W, KROWS)
    cnt3, d23, idx3, any2 = sc(masks_flat, ux, uy, vw)

    cnt_m = cnt3[:, :, 0].T.reshape(MPAD)
    d2_m = d23[:, :, 0].T.reshape(MPAD)
    idx_m = idx3[:, :, 0].T.reshape(MPAD)

    tc = _make_tc_kernel(N, NPAD, M, MPAD, NROW)
    pairs_v, cost = tc(
        cnt_m, v2[:, 0], v2[:, 1],
        cnt_m.reshape(1, MPAD), d2_m.reshape(1, MPAD), idx_m.reshape(1, MPAD),
        any2.reshape(NW, NROW, NPAD // NROW),
        ux.reshape(NROW, NPAD // NROW), uy.reshape(NROW, NPAD // NROW))

    pairs_arr = jnp.stack(
        [pairs_v[0, :M], jnp.arange(M, dtype=jnp.int32)], axis=1)
    return (pairs_arr, cost[0, 0])


# speedup vs baseline: 8.5734x; 1.1616x over previous
import functools

import jax
import jax.numpy as jnp
from jax import lax
from jax.experimental import pallas as pl
from jax.experimental.pallas import tpu as pltpu
from jax.experimental.pallas import tpu_sc as plsc

W_DIRECT = 1.0
W_MULTIPLE = 1.0
W_BACKGROUND = 1.0

NC = 2
NS = 16
L = 16
NW = NC * NS

BIG = 1e30
BIGH = 1e29
IBIG = 2**30


def _make_sc_kernel(N, NPAD, M, MPAD, H, W, KROWS):
    HW = H * W
    WSHIFT = W.bit_length() - 1
    assert W == (1 << WSHIFT)
    CHUNKS = NPAD // L
    REAL_CHUNKS = N // L
    KMAX = (M + NW - 1) // NW
    mesh = plsc.VectorSubcoreMesh(core_axis_name="c", subcore_axis_name="s")

    @functools.partial(
        pl.kernel,
        mesh=mesh,
        compiler_params=pltpu.CompilerParams(needs_layout_passes=False),
        out_type=[
            jax.ShapeDtypeStruct((NW, KROWS, L), jnp.float32),
            jax.ShapeDtypeStruct((NW, KROWS, L), jnp.float32),
            jax.ShapeDtypeStruct((NW, KROWS, L), jnp.int32),
            jax.ShapeDtypeStruct((NW, NPAD), jnp.float32),
        ],
        scratch_types=[
            pltpu.VMEM((NPAD,), jnp.int32),
            pltpu.VMEM((NPAD,), jnp.int32),
            pltpu.VMEM((NPAD,), jnp.int32),
            pltpu.VMEM((NPAD,), jnp.float32),
            pltpu.VMEM((NPAD,), jnp.float32),
            pltpu.VMEM((NPAD,), jnp.float32),
            pltpu.VMEM((NW, L), jnp.float32),
            pltpu.VMEM((KROWS, L), jnp.float32),
            pltpu.VMEM((KROWS, L), jnp.float32),
            pltpu.VMEM((KROWS, L), jnp.int32),
            pltpu.VMEM((L,), jnp.float32),
            pltpu.VMEM((L,), jnp.int32),
            pltpu.VMEM((L,), jnp.float32),
            pltpu.SemaphoreType.DMA,
            pltpu.SemaphoreType.DMA,
        ],
    )
    def sc_kernel(masks_hbm, ux_hbm, uy_hbm, v_hbm,
                  cnt_o, d2_o, idx_o, any_o,
                  pbase_v, offs0_v, offs1_v, vals0_v, vals1_v,
                  any_v, v_v, cntb, d2b, idxb, tmpd, tmpi, tmpc,
                  sem0, sem1):
        wid = lax.axis_index("s") * NC + lax.axis_index("c")
        pltpu.sync_copy(ux_hbm, vals0_v)
        pltpu.sync_copy(uy_hbm, vals1_v)
        pltpu.sync_copy(v_hbm, v_v)

        zero16f = jnp.zeros((L,), jnp.float32)
        zero16i = jnp.zeros((L,), jnp.int32)

        def init_body(i, c):
            sl = pl.ds(i * L, L)
            any_v[sl] = zero16f
            xi = jnp.clip(vals0_v[sl].astype(jnp.int32), 0, W - 1)
            yi = jnp.clip(vals1_v[sl].astype(jnp.int32), 0, H - 1)
            pbase_v[sl] = yi * W + xi
            return c
        lax.fori_loop(0, CHUNKS, init_body, 0)

        for k in range(KROWS):
            cntb[k, :] = zero16f
            d2b[k, :] = zero16f
            idxb[k, :] = zero16i

        base_iota = lax.iota(jnp.int32, L)
        vvec = v_v[wid, :]
        offs_bufs = (offs0_v, offs1_v)
        vals_bufs = (vals0_v, vals1_v)
        sems = (sem0, sem1)

        def build_offs(buf, mhw):
            def offs_body(i, c):
                sl = pl.ds(i * L, L)
                buf[sl] = pbase_v[sl] + mhw
                return c
            lax.fori_loop(0, CHUNKS, offs_body, 0)

        def issue(k):
            b = k % 2
            m = k * NW + wid

            @pl.when(m < M)
            def _():
                build_offs(offs_bufs[b], m * HW)
                pltpu.async_copy(
                    masks_hbm.at[offs_bufs[b]], vals_bufs[b], sems[b])

        def process(k):
            b = k % 2
            m = k * NW + wid

            @pl.when(m < M)
            def _():
                pltpu.make_async_copy(
                    masks_hbm.at[offs_bufs[b]], vals_bufs[b], sems[b]).wait()
                vals_v = vals_bufs[b]
                vx = vvec[k]
                vy = vvec[KROWS + k]

                def chunk_body(i, carry):
                    bd2, bidx, cnt = carry
                    sl = pl.ds(i * L, L)
                    val = vals_v[sl]
                    member = val > 0.0
                    p = pbase_v[sl]
                    xf = jnp.bitwise_and(p, W - 1).astype(jnp.float32)
                    yf = lax.shift_right_logical(p, WSHIFT).astype(jnp.float32)
                    dx = xf - vx
                    dy = yf - vy
                    d2 = dx * dx + dy * dy
                    cand = jnp.where(member, d2, BIG)
                    lt = cand < bd2
                    idxv = base_iota + i * L
                    bd2 = jnp.where(lt, cand, bd2)
                    bidx = jnp.where(lt, idxv, bidx)
                    mf = jnp.where(member, 1.0, 0.0)
                    cnt = cnt + mf
                    any_v[sl] = jnp.maximum(any_v[sl], mf)
                    return (bd2, bidx, cnt)

                bd2, bidx, cnt = lax.fori_loop(
                    0, REAL_CHUNKS, chunk_body,
                    (jnp.full((L,), BIG, jnp.float32), zero16i, zero16f))

                tmpd[:] = bd2
                tmpi[:] = bidx
                tmpc[:] = cnt
                for sh in (8, 4, 2, 1):
                    perm = jnp.bitwise_xor(base_iota, sh)
                    od2 = plsc.load_gather(tmpd, [perm])
                    oidx = plsc.load_gather(tmpi, [perm])
                    oc = plsc.load_gather(tmpc, [perm])
                    d2v = tmpd[:]
                    idxv = tmpi[:]
                    better = (od2 < d2v) | ((od2 == d2v) & (oidx < idxv))
                    tmpd[:] = jnp.where(better, od2, d2v)
                    tmpi[:] = jnp.where(better, oidx, idxv)
                    tmpc[:] = tmpc[:] + oc
                cntb[k, :] = tmpc[:]
                d2b[k, :] = tmpd[:]
                idxb[k, :] = tmpi[:]

        issue(0)
        for k in range(KMAX):
            if k + 1 < KMAX:
                issue(k + 1)
            process(k)

        pltpu.sync_copy(cntb, cnt_o.at[wid])
        pltpu.sync_copy(d2b, d2_o.at[wid])
        pltpu.sync_copy(idxb, idx_o.at[wid])
        pltpu.sync_copy(any_v, any_o.at[wid])

    return sc_kernel


def _make_tc_kernel(N, NPAD, M, MPAD, NROW):
    NCOL = NPAD // NROW

    def tc_body(cnt_s, vx_s, vy_s, cnt_v, d2_v, idx_v, any_r, ux_r, uy_r,
                pairs_o, cost_o):
        ux = ux_r[...]
        uy = uy_r[...]
        rr = lax.broadcasted_iota(jnp.int32, (NROW, NCOL), 0)
        cc = lax.broadcasted_iota(jnp.int32, (NROW, NCOL), 1)
        nidx = rr * NCOL + cc

        anyacc = any_r[0]
        for i in range(1, NW):
            anyacc = anyacc + any_r[i]
        avail = jnp.where((anyacc == 0.0) & (nidx < N), 1.0, 0.0)

        miota = lax.broadcasted_iota(jnp.int32, (1, MPAD), 1)
        cntv = cnt_v[...]
        validm = (cntv > 0.0) & (miota < M)
        wgt = jnp.where(cntv == 1.0, W_DIRECT, W_MULTIPLE).astype(jnp.float32)
        cost0 = jnp.sum(jnp.where(validm, jnp.sqrt(d2_v[...]) * wgt, 0.0))
        pairs = jnp.where(validm, idx_v[...], -1)
        nempty = jnp.sum(jnp.where((cntv <= 0.0) & (miota < M), 1, 0))

        def bg_body(j, st):
            availf, prs, cst = st
            cj = cnt_s[j]
            vxj = vx_s[j]
            vyj = vy_s[j]
            dx = ux - vxj
            dy = uy - vyj
            ds2 = dx * dx + dy * dy
            cand = jnp.where(availf > 0.0, ds2, BIG)
            mv = jnp.min(cand)
            do = (cj == 0.0) & (mv < BIGH)
            mi = jnp.min(jnp.where(cand == mv, nidx, IBIG))
            prs = jnp.where((miota == j) & do, mi, prs)
            cst = cst + jnp.where(do, jnp.sqrt(mv) * W_BACKGROUND, 0.0)
            availf = jnp.where((nidx == mi) & do, 0.0, availf)
            return (availf, prs, cst)

        st = (avail, pairs, jnp.float32(0.0))
        _, pairs, bgcost = lax.cond(
            nempty > 0, lambda s: lax.fori_loop(0, M, bg_body, s),
            lambda s: s, st)

        pairs_o[...] = pairs
        cost_o[...] = (cost0 + bgcost).reshape(1, 1)

    smem = pl.BlockSpec(memory_space=pltpu.SMEM)
    vmem = pl.BlockSpec(memory_space=pltpu.VMEM)
    return pl.pallas_call(
        tc_body,
        in_specs=[smem, smem, smem, vmem, vmem, vmem, vmem, vmem, vmem],
        out_specs=[vmem, vmem],
        out_shape=[
            jax.ShapeDtypeStruct((1, MPAD), jnp.int32),
            jax.ShapeDtypeStruct((1, 1), jnp.float32),
        ],
    )


def kernel(pred_points, points, masks):
    N = pred_points.shape[0]
    M, H, W = masks.shape

    NPAD = ((N + NW * L - 1) // (NW * L)) * (NW * L)
    MPAD = 256
    KROWS = 8
    NROW = 8

    ux = jnp.pad(pred_points[:, 0], (0, NPAD - N))
    uy = jnp.pad(pred_points[:, 1], (0, NPAD - N))
    v2 = jnp.pad(points, ((0, MPAD - M), (0, 0)))
    vw = jnp.concatenate(
        [v2[:, 0].reshape(KROWS, NW).T, v2[:, 1].reshape(KROWS, NW).T],
        axis=1)
    masks_flat = masks.reshape(-1)

    sc = _make_sc_kernel(N, NPAD, M, MPAD, H, W, KROWS)
    cnt3, d23, idx3, any2 = sc(masks_flat, ux, uy, vw)

    cnt_m = cnt3[:, :, 0].T.reshape(MPAD)
    d2_m = d23[:, :, 0].T.reshape(MPAD)
    idx_m = idx3[:, :, 0].T.reshape(MPAD)

    tc = _make_tc_kernel(N, NPAD, M, MPAD, NROW)
    pairs_v, cost = tc(
        cnt_m, v2[:, 0], v2[:, 1],
        cnt_m.reshape(1, MPAD), d2_m.reshape(1, MPAD), idx_m.reshape(1, MPAD),
        any2.reshape(NW, NROW, NPAD // NROW),
        ux.reshape(NROW, NPAD // NROW), uy.reshape(NROW, NPAD // NROW))

    pairs_arr = jnp.stack(
        [pairs_v[0, :M], jnp.arange(M, dtype=jnp.int32)], axis=1)
    return (pairs_arr, cost[0, 0])


# speedup vs baseline: 9.0053x; 1.0504x over previous
import functools

import jax
import jax.numpy as jnp
from jax import lax
from jax.experimental import pallas as pl
from jax.experimental.pallas import tpu as pltpu
from jax.experimental.pallas import tpu_sc as plsc

W_DIRECT = 1.0
W_MULTIPLE = 1.0
W_BACKGROUND = 1.0

NC = 2
NS = 16
L = 16
NW = NC * NS

BIG = 1e30
BIGH = 1e29
IBIG = 2**30


def _make_sc_kernel(N, NPAD, M, MPAD, H, W, KROWS):
    HW = H * W
    WSHIFT = W.bit_length() - 1
    assert W == (1 << WSHIFT)
    KSHIFT = 11
    CMASK = (1 << KSHIFT) - 1
    assert NPAD // L <= CMASK + 1
    assert 2 * (W - 1) * (W - 1) + 1 << KSHIFT < IBIG
    CHUNKS = NPAD // L
    REAL_CHUNKS = N // L
    KMAX = (M + NW - 1) // NW
    mesh = plsc.VectorSubcoreMesh(core_axis_name="c", subcore_axis_name="s")

    @functools.partial(
        pl.kernel,
        mesh=mesh,
        compiler_params=pltpu.CompilerParams(needs_layout_passes=False),
        out_type=[
            jax.ShapeDtypeStruct((NW, KROWS, L), jnp.float32),
            jax.ShapeDtypeStruct((NW, KROWS, L), jnp.float32),
            jax.ShapeDtypeStruct((NW, KROWS, L), jnp.int32),
            jax.ShapeDtypeStruct((NW, NPAD), jnp.float32),
        ],
        scratch_types=[
            pltpu.VMEM((NPAD,), jnp.int32),
            pltpu.VMEM((NPAD,), jnp.int32),
            pltpu.VMEM((NPAD,), jnp.int32),
            pltpu.VMEM((NPAD,), jnp.float32),
            pltpu.VMEM((NPAD,), jnp.float32),
            pltpu.VMEM((NPAD,), jnp.float32),
            pltpu.VMEM((NW, L), jnp.int32),
            pltpu.VMEM((KROWS, L), jnp.float32),
            pltpu.VMEM((KROWS, L), jnp.float32),
            pltpu.VMEM((KROWS, L), jnp.int32),
            pltpu.VMEM((L,), jnp.int32),
            pltpu.VMEM((L,), jnp.int32),
            pltpu.VMEM((L,), jnp.float32),
            pltpu.SemaphoreType.DMA,
            pltpu.SemaphoreType.DMA,
        ],
    )
    def sc_kernel(masks_hbm, ux_hbm, uy_hbm, v_hbm,
                  cnt_o, d2_o, idx_o, any_o,
                  pbase_v, offs0_v, offs1_v, vals0_v, vals1_v,
                  any_v, v_v, cntb, d2b, idxb, tmpd, tmpi, tmpc,
                  sem0, sem1):
        wid = lax.axis_index("s") * NC + lax.axis_index("c")
        pltpu.sync_copy(ux_hbm, vals0_v)
        pltpu.sync_copy(uy_hbm, vals1_v)
        pltpu.sync_copy(v_hbm, v_v)

        zero16f = jnp.zeros((L,), jnp.float32)
        zero16i = jnp.zeros((L,), jnp.int32)

        def init_body(i):
            sl = pl.ds(i * L, L)
            any_v[sl] = zero16f
            xi = jnp.clip(vals0_v[sl].astype(jnp.int32), 0, W - 1)
            yi = jnp.clip(vals1_v[sl].astype(jnp.int32), 0, H - 1)
            pbase_v[sl] = yi * W + xi
        plsc.parallel_loop(0, CHUNKS, unroll=8)(init_body)

        for k in range(KROWS):
            cntb[k, :] = zero16f
            d2b[k, :] = zero16f
            idxb[k, :] = zero16i

        base_iota = lax.iota(jnp.int32, L)
        vvec = v_v[wid, :]
        offs_bufs = (offs0_v, offs1_v)
        vals_bufs = (vals0_v, vals1_v)
        sems = (sem0, sem1)

        def build_offs(buf, mhw):
            def offs_body(i):
                sl = pl.ds(i * L, L)
                buf[sl] = pbase_v[sl] + mhw
            plsc.parallel_loop(0, CHUNKS, unroll=8)(offs_body)

        def issue(k):
            b = k % 2
            m = k * NW + wid

            @pl.when(m < M)
            def _():
                build_offs(offs_bufs[b], m * HW)
                pltpu.async_copy(
                    masks_hbm.at[offs_bufs[b]], vals_bufs[b], sems[b])

        def process(k):
            b = k % 2
            m = k * NW + wid

            @pl.when(m < M)
            def _():
                pltpu.make_async_copy(
                    masks_hbm.at[offs_bufs[b]], vals_bufs[b], sems[b]).wait()
                vals_v = vals_bufs[b]
                vxi = vvec[k]
                vyi = vvec[KROWS + k]

                def chunk_loop(i, carry):
                    bkey, cnt = carry
                    sl = pl.ds(i * L, L)
                    val = vals_v[sl]
                    member = val > 0.0
                    p = pbase_v[sl]
                    xi = jnp.bitwise_and(p, W - 1)
                    yi = lax.shift_right_logical(p, WSHIFT)
                    dxi = xi - vxi
                    dyi = yi - vyi
                    d2i = dxi * dxi + dyi * dyi
                    key = jnp.bitwise_or(lax.shift_left(d2i, KSHIFT), i)
                    key = jnp.where(member, key, IBIG)
                    bkey = jnp.minimum(bkey, key)
                    mf = jnp.where(member, 1.0, 0.0)
                    cnt = cnt + mf
                    any_v[sl] = jnp.maximum(any_v[sl], mf)
                    return (bkey, cnt)

                bkey, cnt = plsc.parallel_loop(
                    0, REAL_CHUNKS, unroll=8,
                    carry=(jnp.full((L,), IBIG, jnp.int32), zero16f),
                )(chunk_loop)

                tmpd[:] = bkey
                tmpi[:] = base_iota
                tmpc[:] = cnt
                for sh in (8, 4, 2, 1):
                    perm = jnp.bitwise_xor(base_iota, sh)
                    okey = plsc.load_gather(tmpd, [perm])
                    og = plsc.load_gather(tmpi, [perm])
                    oc = plsc.load_gather(tmpc, [perm])
                    keyv = tmpd[:]
                    gv = tmpi[:]
                    better = (okey < keyv) | ((okey == keyv) & (og < gv))
                    tmpd[:] = jnp.where(better, okey, keyv)
                    tmpi[:] = jnp.where(better, og, gv)
                    tmpc[:] = tmpc[:] + oc
                keyv = tmpd[:]
                cntb[k, :] = tmpc[:]
                d2b[k, :] = lax.shift_right_logical(keyv, KSHIFT).astype(
                    jnp.float32)
                idxb[k, :] = jnp.bitwise_and(keyv, CMASK) * L + tmpi[:]

        issue(0)
        for k in range(KMAX):
            if k + 1 < KMAX:
                issue(k + 1)
            process(k)

        pltpu.sync_copy(cntb, cnt_o.at[wid])
        pltpu.sync_copy(d2b, d2_o.at[wid])
        pltpu.sync_copy(idxb, idx_o.at[wid])
        pltpu.sync_copy(any_v, any_o.at[wid])

    return sc_kernel


def _make_tc_kernel(N, NPAD, M, MPAD, NROW):
    NCOL = NPAD // NROW

    def tc_body(cnt_s, vx_s, vy_s, cnt_v, d2_v, idx_v, any_r, ux_r, uy_r,
                pairs_o, cost_o):
        ux = ux_r[...]
        uy = uy_r[...]
        rr = lax.broadcasted_iota(jnp.int32, (NROW, NCOL), 0)
        cc = lax.broadcasted_iota(jnp.int32, (NROW, NCOL), 1)
        nidx = rr * NCOL + cc

        anyacc = any_r[0]
        for i in range(1, NW):
            anyacc = anyacc + any_r[i]
        avail = jnp.where((anyacc == 0.0) & (nidx < N), 1.0, 0.0)

        miota = lax.broadcasted_iota(jnp.int32, (1, MPAD), 1)
        cntv = cnt_v[...]
        validm = (cntv > 0.0) & (miota < M)
        wgt = jnp.where(cntv == 1.0, W_DIRECT, W_MULTIPLE).astype(jnp.float32)
        cost0 = jnp.sum(jnp.where(validm, jnp.sqrt(d2_v[...]) * wgt, 0.0))
        pairs = jnp.where(validm, idx_v[...], -1)
        nempty = jnp.sum(jnp.where((cntv <= 0.0) & (miota < M), 1, 0))

        def bg_body(j, st):
            availf, prs, cst = st
            cj = cnt_s[j]
            vxj = vx_s[j]
            vyj = vy_s[j]
            dx = ux - vxj
            dy = uy - vyj
            ds2 = dx * dx + dy * dy
            cand = jnp.where(availf > 0.0, ds2, BIG)
            mv = jnp.min(cand)
            do = (cj == 0.0) & (mv < BIGH)
            mi = jnp.min(jnp.where(cand == mv, nidx, IBIG))
            prs = jnp.where((miota == j) & do, mi, prs)
            cst = cst + jnp.where(do, jnp.sqrt(mv) * W_BACKGROUND, 0.0)
            availf = jnp.where((nidx == mi) & do, 0.0, availf)
            return (availf, prs, cst)

        st = (avail, pairs, jnp.float32(0.0))
        _, pairs, bgcost = lax.cond(
            nempty > 0, lambda s: lax.fori_loop(0, M, bg_body, s),
            lambda s: s, st)

        pairs_o[...] = pairs
        cost_o[...] = (cost0 + bgcost).reshape(1, 1)

    smem = pl.BlockSpec(memory_space=pltpu.SMEM)
    vmem = pl.BlockSpec(memory_space=pltpu.VMEM)
    return pl.pallas_call(
        tc_body,
        in_specs=[smem, smem, smem, vmem, vmem, vmem, vmem, vmem, vmem],
        out_specs=[vmem, vmem],
        out_shape=[
            jax.ShapeDtypeStruct((1, MPAD), jnp.int32),
            jax.ShapeDtypeStruct((1, 1), jnp.float32),
        ],
    )


def kernel(pred_points, points, masks):
    N = pred_points.shape[0]
    M, H, W = masks.shape

    NPAD = ((N + NW * L - 1) // (NW * L)) * (NW * L)
    MPAD = 256
    KROWS = 8
    NROW = 8

    ux = jnp.pad(pred_points[:, 0], (0, NPAD - N))
    uy = jnp.pad(pred_points[:, 1], (0, NPAD - N))
    v2 = jnp.pad(points, ((0, MPAD - M), (0, 0)))
    vw = jnp.concatenate(
        [v2[:, 0].reshape(KROWS, NW).T, v2[:, 1].reshape(KROWS, NW).T],
        axis=1).astype(jnp.int32)
    masks_flat = masks.reshape(-1)

    sc = _make_sc_kernel(N, NPAD, M, MPAD, H, W, KROWS)
    cnt3, d23, idx3, any2 = sc(masks_flat, ux, uy, vw)

    cnt_m = cnt3[:, :, 0].T.reshape(MPAD)
    d2_m = d23[:, :, 0].T.reshape(MPAD)
    idx_m = idx3[:, :, 0].T.reshape(MPAD)

    tc = _make_tc_kernel(N, NPAD, M, MPAD, NROW)
    pairs_v, cost = tc(
        cnt_m, v2[:, 0], v2[:, 1],
        cnt_m.reshape(1, MPAD), d2_m.reshape(1, MPAD), idx_m.reshape(1, MPAD),
        any2.reshape(NW, NROW, NPAD // NROW),
        ux.reshape(NROW, NPAD // NROW), uy.reshape(NROW, NPAD // NROW))

    pairs_arr = jnp.stack(
        [pairs_v[0, :M], jnp.arange(M, dtype=jnp.int32)], axis=1)
    return (pairs_arr, cost[0, 0])
